# baseline (speedup 1.0000x reference)
import functools

import jax
import jax.numpy as jnp
from jax.experimental import pallas as pl


def _moe_kernel(x_ref, cp_ref, w1_ref, b1_ref, w2bd_ref, b2_ref,
                pred_ref, gates_ref, used_ref, infl_ref, act_ref):
    i = pl.program_id(0)

    xb = x_ref[...].astype(jnp.bfloat16)
    h = jnp.dot(xb, w1_ref[...], preferred_element_type=jnp.float32)
    h = jnp.maximum(h + b1_ref[...], 0.0)
    s = jnp.dot(h, w2bd_ref[...], preferred_element_type=jnp.float32)
    out_e = jax.nn.sigmoid(s + b2_ref[...])

    cp = cp_ref[...]
    m = jnp.max(cp, axis=1, keepdims=True)
    eg = jnp.exp(cp - m)
    gates = eg / jnp.sum(eg, axis=1, keepdims=True)

    pred_ref[...] = jnp.sum(gates * out_e, axis=1, keepdims=True)
    gates_ref[...] = gates
    hot = (gates > 0.01).astype(jnp.float32)
    used_ref[...] = jnp.sum(hot, axis=1, keepdims=True)

    infl = jnp.sum(gates, axis=0, keepdims=True)
    act = jnp.sum(hot, axis=0, keepdims=True)

    @pl.when(i == 0)
    def _():
        infl_ref[...] = infl
        act_ref[...] = act

    @pl.when(i != 0)
    def _():
        infl_ref[...] += infl
        act_ref[...] += act


@functools.partial(jax.jit, static_argnames=())
def kernel(x, cluster_probs, W1, b1, W2, b2):
    B, D = x.shape
    E, _, H = W1.shape
    EH = E * H
    R = 512

    w1r = jnp.transpose(W1, (1, 0, 2)).reshape(D, EH).astype(jnp.bfloat16)
    b1r = b1.reshape(1, EH)
    eye = (jnp.arange(EH)[:, None] // H == jnp.arange(E)[None, :])
    w2bd = jnp.where(eye, W2.reshape(EH, 1), 0.0)
    b2r = b2.reshape(1, E)

    grid = (B // R,)
    out = pl.pallas_call(
        _moe_kernel,
        grid=grid,
        in_specs=[
            pl.BlockSpec((R, D), lambda i: (i, 0)),
            pl.BlockSpec((R, E), lambda i: (i, 0)),
            pl.BlockSpec((D, EH), lambda i: (0, 0)),
            pl.BlockSpec((1, EH), lambda i: (0, 0)),
            pl.BlockSpec((EH, E), lambda i: (0, 0)),
            pl.BlockSpec((1, E), lambda i: (0, 0)),
        ],
        out_specs=[
            pl.BlockSpec((R, 1), lambda i: (i, 0)),
            pl.BlockSpec((R, E), lambda i: (i, 0)),
            pl.BlockSpec((R, 1), lambda i: (i, 0)),
            pl.BlockSpec((1, E), lambda i: (0, 0)),
            pl.BlockSpec((1, E), lambda i: (0, 0)),
        ],
        out_shape=[
            jax.ShapeDtypeStruct((B, 1), jnp.float32),
            jax.ShapeDtypeStruct((B, E), jnp.float32),
            jax.ShapeDtypeStruct((B, 1), jnp.float32),
            jax.ShapeDtypeStruct((1, E), jnp.float32),
            jax.ShapeDtypeStruct((1, E), jnp.float32),
        ],
    )(x, cluster_probs, w1r, b1r, w2bd, b2r)

    prediction, gates, used, infl, act = out
    return (prediction, gates, used.reshape(B), infl.reshape(E), act.reshape(E))


# speedup vs baseline: 1.1226x; 1.1226x over previous
import functools

import jax
import jax.numpy as jnp
from jax.experimental import pallas as pl


def _moe_kernel(x_ref, cp_ref, w1_ref, b1_ref, w2bd_ref, b2_ref,
                pred_ref, gates_ref, used_ref, infl_ref, act_ref):
    i = pl.program_id(0)

    xb = x_ref[...].astype(jnp.bfloat16)
    h = jnp.dot(xb, w1_ref[...], preferred_element_type=jnp.float32)
    h = jnp.maximum(h + b1_ref[...], 0.0)
    s = jnp.dot(h, w2bd_ref[...], preferred_element_type=jnp.float32)
    out_e = jax.nn.sigmoid(s + b2_ref[...])

    cp = cp_ref[...]
    m = jnp.max(cp, axis=1, keepdims=True)
    eg = jnp.exp(cp - m)
    gates = eg / jnp.sum(eg, axis=1, keepdims=True)

    pred_ref[...] = jnp.sum(gates * out_e, axis=1, keepdims=True)
    gates_ref[...] = gates
    hot = (gates > 0.01).astype(jnp.float32)
    used_ref[...] = jnp.sum(hot, axis=1, keepdims=True)

    infl = jnp.sum(gates, axis=0, keepdims=True)
    act = jnp.sum(hot, axis=0, keepdims=True)

    @pl.when(i == 0)
    def _():
        infl_ref[...] = infl
        act_ref[...] = act

    @pl.when(i != 0)
    def _():
        infl_ref[...] += infl
        act_ref[...] += act


@functools.partial(jax.jit, static_argnames=())
def kernel(x, cluster_probs, W1, b1, W2, b2):
    B, D = x.shape
    E, _, H = W1.shape
    EH = E * H
    R = 1024

    w1r = jnp.transpose(W1, (1, 0, 2)).reshape(D, EH).astype(jnp.bfloat16)
    b1r = b1.reshape(1, EH)
    eye = (jnp.arange(EH)[:, None] // H == jnp.arange(E)[None, :])
    w2bd = jnp.where(eye, W2.reshape(EH, 1), 0.0)
    b2r = b2.reshape(1, E)

    grid = (B // R,)
    out = pl.pallas_call(
        _moe_kernel,
        grid=grid,
        in_specs=[
            pl.BlockSpec((R, D), lambda i: (i, 0)),
            pl.BlockSpec((R, E), lambda i: (i, 0)),
            pl.BlockSpec((D, EH), lambda i: (0, 0)),
            pl.BlockSpec((1, EH), lambda i: (0, 0)),
            pl.BlockSpec((EH, E), lambda i: (0, 0)),
            pl.BlockSpec((1, E), lambda i: (0, 0)),
        ],
        out_specs=[
            pl.BlockSpec((R, 1), lambda i: (i, 0)),
            pl.BlockSpec((R, E), lambda i: (i, 0)),
            pl.BlockSpec((R, 1), lambda i: (i, 0)),
            pl.BlockSpec((1, E), lambda i: (0, 0)),
            pl.BlockSpec((1, E), lambda i: (0, 0)),
        ],
        out_shape=[
            jax.ShapeDtypeStruct((B, 1), jnp.float32),
            jax.ShapeDtypeStruct((B, E), jnp.float32),
            jax.ShapeDtypeStruct((B, 1), jnp.float32),
            jax.ShapeDtypeStruct((1, E), jnp.float32),
            jax.ShapeDtypeStruct((1, E), jnp.float32),
        ],
    )(x, cluster_probs, w1r, b1r, w2bd, b2r)

    prediction, gates, used, infl, act = out
    return (prediction, gates, used.reshape(B), infl.reshape(E), act.reshape(E))


# speedup vs baseline: 1.1519x; 1.0261x over previous
import functools

import jax
import jax.numpy as jnp
from jax.experimental import pallas as pl


def _moe_kernel(x_ref, cp_ref, w1_ref, b1_ref, w2bd_ref, b2_ref,
                pred_ref, gates_ref, used_ref, infl_ref, act_ref):
    i = pl.program_id(0)

    xb = x_ref[...].astype(jnp.bfloat16)
    h = jnp.dot(xb, w1_ref[...], preferred_element_type=jnp.float32)
    h = jnp.maximum(h + b1_ref[...], 0.0)
    s = jnp.dot(h, w2bd_ref[...], preferred_element_type=jnp.float32)
    out_e = jax.nn.sigmoid(s + b2_ref[...])

    cp = cp_ref[...]
    m = jnp.max(cp, axis=1, keepdims=True)
    eg = jnp.exp(cp - m)
    gates = eg / jnp.sum(eg, axis=1, keepdims=True)

    pred_ref[...] = jnp.sum(gates * out_e, axis=1, keepdims=True)
    gates_ref[...] = gates
    hot = (gates > 0.01).astype(jnp.float32)
    used_ref[...] = jnp.sum(hot, axis=1, keepdims=True)

    infl = jnp.sum(gates, axis=0, keepdims=True)
    act = jnp.sum(hot, axis=0, keepdims=True)

    @pl.when(i == 0)
    def _():
        infl_ref[...] = infl
        act_ref[...] = act

    @pl.when(i != 0)
    def _():
        infl_ref[...] += infl
        act_ref[...] += act


@functools.partial(jax.jit, static_argnames=())
def kernel(x, cluster_probs, W1, b1, W2, b2):
    B, D = x.shape
    E, _, H = W1.shape
    EH = E * H
    R = 2048

    w1r = jnp.transpose(W1, (1, 0, 2)).reshape(D, EH).astype(jnp.bfloat16)
    b1r = b1.reshape(1, EH)
    eye = (jnp.arange(EH)[:, None] // H == jnp.arange(E)[None, :])
    w2bd = jnp.where(eye, W2.reshape(EH, 1), 0.0)
    b2r = b2.reshape(1, E)

    grid = (B // R,)
    out = pl.pallas_call(
        _moe_kernel,
        grid=grid,
        in_specs=[
            pl.BlockSpec((R, D), lambda i: (i, 0)),
            pl.BlockSpec((R, E), lambda i: (i, 0)),
            pl.BlockSpec((D, EH), lambda i: (0, 0)),
            pl.BlockSpec((1, EH), lambda i: (0, 0)),
            pl.BlockSpec((EH, E), lambda i: (0, 0)),
            pl.BlockSpec((1, E), lambda i: (0, 0)),
        ],
        out_specs=[
            pl.BlockSpec((R, 1), lambda i: (i, 0)),
            pl.BlockSpec((R, E), lambda i: (i, 0)),
            pl.BlockSpec((R, 1), lambda i: (i, 0)),
            pl.BlockSpec((1, E), lambda i: (0, 0)),
            pl.BlockSpec((1, E), lambda i: (0, 0)),
        ],
        out_shape=[
            jax.ShapeDtypeStruct((B, 1), jnp.float32),
            jax.ShapeDtypeStruct((B, E), jnp.float32),
            jax.ShapeDtypeStruct((B, 1), jnp.float32),
            jax.ShapeDtypeStruct((1, E), jnp.float32),
            jax.ShapeDtypeStruct((1, E), jnp.float32),
        ],
    )(x, cluster_probs, w1r, b1r, w2bd, b2r)

    prediction, gates, used, infl, act = out
    return (prediction, gates, used.reshape(B), infl.reshape(E), act.reshape(E))
